# baseline (speedup 1.0000x reference)
import functools

import jax
import jax.numpy as jnp
from jax import lax
from jax.experimental import pallas as pl
from jax.experimental.pallas import tpu as pltpu
from jax.experimental.pallas import tpu_sc as plsc

B = 16384
D = 32
J = 6
NC = 2
NS = 16
NW = NC * NS
RPW = B // NW
CHUNK = 128
U_CHUNKS = RPW // CHUNK
V_CHUNKS = RPW * J // CHUNK
GROUPS = RPW // 16


def _sc_body(idx_u_hbm, idx_v_hbm, u_hbm, v_hbm, out_hbm,
             idxu_v, idxv_v, urows_v, vrows_v, pred_v, sem):
    wid = lax.axis_index("s") * NC + lax.axis_index("c")
    base = wid * RPW

    pltpu.sync_copy(idx_u_hbm.at[pl.ds(base, RPW)], idxu_v)
    pltpu.sync_copy(idx_v_hbm.at[pl.ds(base * J, RPW * J)], idxv_v)

    for c in range(U_CHUNKS):
        pltpu.make_async_copy(
            u_hbm.at[idxu_v.at[pl.ds(c * CHUNK, CHUNK)]],
            urows_v.at[pl.ds(c * CHUNK, CHUNK)], sem).start()

    def fire_v(c, carry):
        pltpu.make_async_copy(
            v_hbm.at[idxv_v.at[pl.ds(c * CHUNK, CHUNK)]],
            vrows_v.at[pl.ds(c * CHUNK, CHUNK)], sem).start()
        return carry
    lax.fori_loop(0, V_CHUNKS, fire_v, 0)

    for c in range(U_CHUNKS):
        pltpu.make_async_copy(
            u_hbm.at[idxu_v.at[pl.ds(c * CHUNK, CHUNK)]],
            urows_v.at[pl.ds(c * CHUNK, CHUNK)], sem).wait()

    def drain_v(c, carry):
        pltpu.make_async_copy(
            v_hbm.at[idxv_v.at[pl.ds(c * CHUNK, CHUNK)]],
            vrows_v.at[pl.ds(c * CHUNK, CHUNK)], sem).wait()
        return carry
    lax.fori_loop(0, V_CHUNKS, drain_v, 0)

    lanes = lax.iota(jnp.int32, 16)
    zero = jnp.zeros((16,), jnp.float32)

    def group_body(g, carry):
        rows_u = g * 16 + lanes

        def d_body(d, accs):
            dcol = jnp.zeros((16,), jnp.int32) + d
            uvec = plsc.load_gather(urows_v, [rows_u, dcol])
            return tuple(
                accs[j] + uvec * plsc.load_gather(
                    vrows_v, [rows_u * J + j, dcol])
                for j in range(J))

        accs = lax.fori_loop(0, D, d_body, (zero,) * J)
        for j in range(J):
            pred_v[j, pl.ds(g * 16, 16)] = accs[j]
        return carry
    lax.fori_loop(0, GROUPS, group_body, 0)

    pltpu.sync_copy(pred_v, out_hbm.at[wid])


_sc_pred = functools.partial(
    pl.kernel,
    mesh=plsc.VectorSubcoreMesh(core_axis_name="c", subcore_axis_name="s"),
    out_type=jax.ShapeDtypeStruct((NW, J, RPW), jnp.float32),
    scratch_types=[
        pltpu.VMEM((RPW,), jnp.int32),
        pltpu.VMEM((RPW * J,), jnp.int32),
        pltpu.VMEM((RPW, D), jnp.float32),
        pltpu.VMEM((RPW * J, D), jnp.float32),
        pltpu.VMEM((J, RPW), jnp.float32),
        pltpu.SemaphoreType.DMA,
    ],
    compiler_params=pltpu.CompilerParams(
        needs_layout_passes=False,
        use_tc_tiling_on_sc=False,
    ),
)(_sc_body)


def _tc_loss_body(x_ref, o_ref):
    x = x_ref[...]
    ls = jnp.minimum(x, 0.0) - jnp.log(1.0 + jnp.exp(-jnp.abs(x)))
    rows = lax.broadcasted_iota(jnp.int32, x.shape, 0) % J
    w = jnp.where(rows == 0, 1.0, -1.0)
    o_ref[0, 0] = jnp.sum(w * ls)


def kernel(pos_u, pos_neg_v, U, V):
    idx_u = pos_u.reshape(B)
    idx_v = pos_neg_v.reshape(B * J)
    pred = _sc_pred(idx_u, idx_v, U, V)
    loss2d = pl.pallas_call(
        _tc_loss_body,
        out_shape=jax.ShapeDtypeStruct((1, 1), jnp.float32),
        out_specs=pl.BlockSpec(memory_space=pltpu.SMEM),
    )(pred.reshape(NW * J, RPW))
    return loss2d[0, 0]
